# baseline (speedup 1.0000x reference)
import functools

import jax
import jax.numpy as jnp
from jax import lax
from jax.experimental import pallas as pl
from jax.experimental.pallas import tpu as pltpu
from jax.experimental.pallas import tpu_sc as plsc

N_NODES = 10000
N_EDGES = 320000
D = 128

NC = 2
NS = 16
NW = NC * NS

CHUNK = 128
NCHUNK = 80
E_TILE = CHUNK * NCHUNK
E_PAD = E_TILE * NW

AGG_ROWS = 10112
ROWS_PER_TILE = AGG_ROWS // NS


def _unpack_chunk(pk_v, j, src_b, dst_b):
    for r in range(CHUNK // 16):
        v = pk_v[j, pl.ds(16 * r, 16)]
        src_b[pl.ds(16 * r, 16)] = lax.bitwise_and(v, 0xFFFF)
        dst_b[pl.ds(16 * r, 16)] = lax.shift_right_logical(v, 16)


def _sc_agg_body(x_hbm, pk_hbm, z_hbm, p_hbm,
                 pk_v, srcb0, dstb0, srcb1, dstb1, gbuf0, gbuf1, agg,
                 sem0, sem1):
    cid = lax.axis_index("c")
    sid = lax.axis_index("s")
    wid = cid * NS + sid

    pltpu.sync_copy(z_hbm.at[pl.ds(sid * ROWS_PER_TILE, ROWS_PER_TILE)],
                    agg.at[pl.ds(sid * ROWS_PER_TILE, ROWS_PER_TILE)])
    pltpu.sync_copy(pk_hbm.at[wid], pk_v)
    plsc.subcore_barrier()

    _unpack_chunk(pk_v, 0, srcb0, dstb0)
    pltpu.async_copy(x_hbm.at[srcb0], gbuf0, sem0)

    def body(i, carry):
        j = 2 * i
        jn = jnp.minimum(j + 2, NCHUNK - 1)
        _unpack_chunk(pk_v, j + 1, srcb1, dstb1)
        pltpu.async_copy(x_hbm.at[srcb1], gbuf1, sem1)
        pltpu.make_async_copy(x_hbm.at[srcb0], gbuf0, sem0).wait()
        pltpu.sync_copy(gbuf0, agg.at[dstb0], add=True)
        _unpack_chunk(pk_v, jn, srcb0, dstb0)
        pltpu.async_copy(x_hbm.at[srcb0], gbuf0, sem0)
        pltpu.make_async_copy(x_hbm.at[srcb1], gbuf1, sem1).wait()
        pltpu.sync_copy(gbuf1, agg.at[dstb1], add=True)
        return carry

    lax.fori_loop(0, NCHUNK // 2, body, 0)
    pltpu.make_async_copy(x_hbm.at[srcb0], gbuf0, sem0).wait()
    plsc.subcore_barrier()

    pltpu.sync_copy(agg.at[pl.ds(sid * ROWS_PER_TILE, ROWS_PER_TILE)],
                    p_hbm.at[cid, pl.ds(sid * ROWS_PER_TILE, ROWS_PER_TILE)])


def _sc_aggregate(x, packed, zeros):
    mesh = plsc.VectorSubcoreMesh(core_axis_name="c", subcore_axis_name="s")
    f = functools.partial(
        pl.kernel,
        mesh=mesh,
        out_type=jax.ShapeDtypeStruct((NC, AGG_ROWS, D), jnp.float32),
        scratch_types=[
            pltpu.VMEM((NCHUNK, CHUNK), jnp.int32),
            pltpu.VMEM((CHUNK,), jnp.int32),
            pltpu.VMEM((CHUNK,), jnp.int32),
            pltpu.VMEM((CHUNK,), jnp.int32),
            pltpu.VMEM((CHUNK,), jnp.int32),
            pltpu.VMEM((CHUNK, D), jnp.float32),
            pltpu.VMEM((CHUNK, D), jnp.float32),
            pltpu.VMEM_SHARED((AGG_ROWS, D), jnp.float32),
            pltpu.SemaphoreType.DMA,
            pltpu.SemaphoreType.DMA,
        ],
    )(_sc_agg_body)
    return f(x, packed, zeros)


def _mlp_body(eps_ref, x_ref, p0_ref, p1_ref, w1_ref, b1_ref, w2_ref, b2_ref,
              o_ref):
    h = (1.0 + eps_ref[0, 0]) * x_ref[...] + p0_ref[...] + p1_ref[...]
    h = jnp.dot(h, w1_ref[...], preferred_element_type=jnp.float32) + b1_ref[...]
    h = jnp.maximum(h, 0.0)
    h = jnp.dot(h, w2_ref[...], preferred_element_type=jnp.float32) + b2_ref[...]
    o_ref[...] = jnp.maximum(h, 0.0)


def _mlp(x, p0, p1, W1, b1, W2, b2, eps):
    blk = 1000
    grid = (N_NODES // blk,)
    row_spec = pl.BlockSpec((blk, D), lambda i: (i, 0))
    full_spec = pl.BlockSpec((D, D), lambda i: (0, 0))
    bias_spec = pl.BlockSpec((1, D), lambda i: (0, 0))
    return pl.pallas_call(
        _mlp_body,
        grid=grid,
        in_specs=[
            pl.BlockSpec(memory_space=pltpu.SMEM),
            row_spec, row_spec, row_spec,
            full_spec, bias_spec, full_spec, bias_spec,
        ],
        out_specs=row_spec,
        out_shape=jax.ShapeDtypeStruct((N_NODES, D), jnp.float32),
    )(eps.reshape(1, 1), x, p0, p1, W1, b1.reshape(1, D), W2, b2.reshape(1, D))


def kernel(x, edge_index, W1, b1, W2, b2, eps):
    src = edge_index[0].astype(jnp.int32)
    dst = edge_index[1].astype(jnp.int32)
    pad = E_PAD - N_EDGES
    packed = jnp.left_shift(dst, 16) | src
    packed = jnp.concatenate(
        [packed, jnp.full((pad,), N_NODES << 16, jnp.int32)]).reshape(
            NW, NCHUNK, CHUNK)
    zeros = jnp.zeros((AGG_ROWS, D), jnp.float32)

    p = _sc_aggregate(x, packed, zeros)
    return _mlp(x, p[0, :N_NODES], p[1, :N_NODES], W1, b1, W2, b2, eps)


# speedup vs baseline: 1.0001x; 1.0001x over previous
import functools

import jax
import jax.numpy as jnp
from jax import lax
from jax.experimental import pallas as pl
from jax.experimental.pallas import tpu as pltpu
from jax.experimental.pallas import tpu_sc as plsc

N_NODES = 10000
N_EDGES = 320000
D = 128

NC = 2
NS = 16
NW = NC * NS

CHUNK = 128
NCHUNK = 80
E_TILE = CHUNK * NCHUNK
E_PAD = E_TILE * NW

AGG_ROWS = 10112
ROWS_PER_TILE = AGG_ROWS // NS


def _unpack_chunk(pk_v, j, src_b, dst_b):
    for r in range(CHUNK // 16):
        v = pk_v[j, pl.ds(16 * r, 16)]
        src_b[pl.ds(16 * r, 16)] = lax.bitwise_and(v, 0xFFFF)
        dst_b[pl.ds(16 * r, 16)] = lax.shift_right_logical(v, 16)


def _sc_agg_body(x_hbm, pk_hbm, z_hbm, p_hbm,
                 pk_v, srcb0, dstb0, srcb1, dstb1, gbuf0, gbuf1, agg,
                 sem0, sem1):
    cid = lax.axis_index("c")
    sid = lax.axis_index("s")
    wid = cid * NS + sid

    pltpu.sync_copy(z_hbm.at[pl.ds(sid * ROWS_PER_TILE, ROWS_PER_TILE)],
                    agg.at[pl.ds(sid * ROWS_PER_TILE, ROWS_PER_TILE)])
    pltpu.sync_copy(pk_hbm.at[wid], pk_v)
    plsc.subcore_barrier()

    _unpack_chunk(pk_v, 0, srcb0, dstb0)
    pltpu.async_copy(x_hbm.at[srcb0], gbuf0, sem0)

    def body(i, carry):
        j = 2 * i
        jn = jnp.minimum(j + 2, NCHUNK - 1)
        _unpack_chunk(pk_v, j + 1, srcb1, dstb1)
        pltpu.async_copy(x_hbm.at[srcb1], gbuf1, sem1)
        pltpu.make_async_copy(x_hbm.at[srcb0], gbuf0, sem0).wait()
        pltpu.sync_copy(gbuf0, agg.at[dstb0], add=True)
        _unpack_chunk(pk_v, jn, srcb0, dstb0)
        pltpu.async_copy(x_hbm.at[srcb0], gbuf0, sem0)
        pltpu.make_async_copy(x_hbm.at[srcb1], gbuf1, sem1).wait()
        pltpu.sync_copy(gbuf1, agg.at[dstb1], add=True)
        return carry

    lax.fori_loop(0, NCHUNK // 2, body, 0)
    pltpu.make_async_copy(x_hbm.at[srcb0], gbuf0, sem0).wait()
    plsc.subcore_barrier()

    pltpu.sync_copy(agg.at[pl.ds(sid * ROWS_PER_TILE, ROWS_PER_TILE)],
                    p_hbm.at[cid, pl.ds(sid * ROWS_PER_TILE, ROWS_PER_TILE)])


def _sc_aggregate(x, packed, zeros):
    mesh = plsc.VectorSubcoreMesh(core_axis_name="c", subcore_axis_name="s")
    f = functools.partial(
        pl.kernel,
        mesh=mesh,
        out_type=jax.ShapeDtypeStruct((NC, AGG_ROWS, D), jnp.float32),
        scratch_types=[
            pltpu.VMEM((NCHUNK, CHUNK), jnp.int32),
            pltpu.VMEM((CHUNK,), jnp.int32),
            pltpu.VMEM((CHUNK,), jnp.int32),
            pltpu.VMEM((CHUNK,), jnp.int32),
            pltpu.VMEM((CHUNK,), jnp.int32),
            pltpu.VMEM((CHUNK, D), jnp.float32),
            pltpu.VMEM((CHUNK, D), jnp.float32),
            pltpu.VMEM_SHARED((AGG_ROWS, D), jnp.float32),
            pltpu.SemaphoreType.DMA,
            pltpu.SemaphoreType.DMA,
        ],
    )(_sc_agg_body)
    return f(x, packed, zeros)


def _mlp_body(eps_ref, x_ref, p0_ref, p1_ref, w1_ref, b1_ref, w2_ref, b2_ref,
              o_ref):
    h = (1.0 + eps_ref[0, 0]) * x_ref[...] + p0_ref[...] + p1_ref[...]
    h = jnp.dot(h, w1_ref[...], preferred_element_type=jnp.float32) + b1_ref[...]
    h = jnp.maximum(h, 0.0)
    h = jnp.dot(h, w2_ref[...], preferred_element_type=jnp.float32) + b2_ref[...]
    o_ref[...] = jnp.maximum(h, 0.0)


def _mlp(x, p0, p1, W1, b1, W2, b2, eps):
    blk = 1000
    grid = (N_NODES // blk,)
    row_spec = pl.BlockSpec((blk, D), lambda i: (i, 0))
    full_spec = pl.BlockSpec((D, D), lambda i: (0, 0))
    bias_spec = pl.BlockSpec((1, D), lambda i: (0, 0))
    return pl.pallas_call(
        _mlp_body,
        grid=grid,
        in_specs=[
            pl.BlockSpec(memory_space=pltpu.SMEM),
            row_spec, row_spec, row_spec,
            full_spec, bias_spec, full_spec, bias_spec,
        ],
        out_specs=row_spec,
        out_shape=jax.ShapeDtypeStruct((N_NODES, D), jnp.float32),
    )(eps.reshape(1, 1), x, p0, p1, W1, b1.reshape(1, D), W2, b2.reshape(1, D))


def kernel(x, edge_index, W1, b1, W2, b2, eps):
    src = edge_index[0].astype(jnp.int32)
    dst = edge_index[1].astype(jnp.int32)
    pad = E_PAD - N_EDGES
    packed = jnp.left_shift(dst, 16) | src
    trash = (N_NODES + jnp.arange(pad, dtype=jnp.int32)
             % (AGG_ROWS - N_NODES)) << 16
    packed = jnp.concatenate([packed, trash]).reshape(NW, NCHUNK, CHUNK)
    zeros = jnp.zeros((AGG_ROWS, D), jnp.float32)

    p = _sc_aggregate(x, packed, zeros)
    return _mlp(x, p[0, :N_NODES], p[1, :N_NODES], W1, b1, W2, b2, eps)


# speedup vs baseline: 3.1845x; 3.1843x over previous
import functools

import jax
import jax.numpy as jnp
from jax import lax
from jax.experimental import pallas as pl
from jax.experimental.pallas import tpu as pltpu
from jax.experimental.pallas import tpu_sc as plsc

N_NODES = 10000
N_EDGES = 320000
D = 128

NC = 2
NS = 16
NW = NC * NS

CHUNK = 128
NCHUNK = 80
E_TILE = CHUNK * NCHUNK
E_PAD = E_TILE * NW

AGG_ROWS = 10112
ROWS_PER_TILE = AGG_ROWS // NS


def _unpack_chunk(pk_v, j, src_b, dst_b):
    for r in range(CHUNK // 16):
        v = pk_v[j, pl.ds(16 * r, 16)]
        src_b[pl.ds(16 * r, 16)] = lax.bitwise_and(v, 0xFFFF)
        dst_b[pl.ds(16 * r, 16)] = lax.shift_right_logical(v, 16)


def _sc_agg_body(x_hbm, pk_hbm, z_hbm, p_hbm,
                 pk_v, srcb0, dstb0, srcb1, dstb1, gbuf0, gbuf1, agg,
                 sem0, sem1):
    cid = lax.axis_index("c")
    sid = lax.axis_index("s")
    wid = cid * NS + sid

    with jax.named_scope("sc_init"):
        pltpu.sync_copy(z_hbm.at[pl.ds(sid * ROWS_PER_TILE, ROWS_PER_TILE)],
                        agg.at[pl.ds(sid * ROWS_PER_TILE, ROWS_PER_TILE)])
        pltpu.sync_copy(pk_hbm.at[wid], pk_v)
        plsc.subcore_barrier()

    _unpack_chunk(pk_v, 0, srcb0, dstb0)
    pltpu.async_copy(x_hbm.at[srcb0], gbuf0, sem0)

    def body(i, carry):
        j = 2 * i
        jn = jnp.minimum(j + 2, NCHUNK - 1)
        _unpack_chunk(pk_v, j + 1, srcb1, dstb1)
        pltpu.async_copy(x_hbm.at[srcb1], gbuf1, sem1)
        pltpu.make_async_copy(x_hbm.at[srcb0], gbuf0, sem0).wait()
        pltpu.sync_copy(gbuf0, agg.at[dstb0], add=True)
        _unpack_chunk(pk_v, jn, srcb0, dstb0)
        pltpu.async_copy(x_hbm.at[srcb0], gbuf0, sem0)
        pltpu.make_async_copy(x_hbm.at[srcb1], gbuf1, sem1).wait()
        pltpu.sync_copy(gbuf1, agg.at[dstb1], add=True)
        return carry

    with jax.named_scope("sc_edge_loop"):
        lax.fori_loop(0, NCHUNK // 2, body, 0)
        pltpu.make_async_copy(x_hbm.at[srcb0], gbuf0, sem0).wait()
        plsc.subcore_barrier()

    with jax.named_scope("sc_writeback"):
        pltpu.sync_copy(agg.at[pl.ds(sid * ROWS_PER_TILE, ROWS_PER_TILE)],
                        p_hbm.at[cid, pl.ds(sid * ROWS_PER_TILE, ROWS_PER_TILE)])


def _sc_aggregate(x, packed, zeros):
    mesh = plsc.VectorSubcoreMesh(core_axis_name="c", subcore_axis_name="s")
    f = functools.partial(
        pl.kernel,
        mesh=mesh,
        out_type=jax.ShapeDtypeStruct((NC, AGG_ROWS, D), jnp.float32),
        scratch_types=[
            pltpu.VMEM((NCHUNK, CHUNK), jnp.int32),
            pltpu.VMEM((CHUNK,), jnp.int32),
            pltpu.VMEM((CHUNK,), jnp.int32),
            pltpu.VMEM((CHUNK,), jnp.int32),
            pltpu.VMEM((CHUNK,), jnp.int32),
            pltpu.VMEM((CHUNK, D), jnp.float32),
            pltpu.VMEM((CHUNK, D), jnp.float32),
            pltpu.VMEM_SHARED((AGG_ROWS, D), jnp.float32),
            pltpu.SemaphoreType.DMA,
            pltpu.SemaphoreType.DMA,
        ],
    )(_sc_agg_body)
    return f(x, packed, zeros)


def _mlp_body(eps_ref, x_ref, p0_ref, p1_ref, w1_ref, b1_ref, w2_ref, b2_ref,
              o_ref):
    h = (1.0 + eps_ref[0, 0]) * x_ref[...] + p0_ref[...] + p1_ref[...]
    h = jnp.dot(h, w1_ref[...], preferred_element_type=jnp.float32) + b1_ref[...]
    h = jnp.maximum(h, 0.0)
    h = jnp.dot(h, w2_ref[...], preferred_element_type=jnp.float32) + b2_ref[...]
    o_ref[...] = jnp.maximum(h, 0.0)


def _mlp(x, p0, p1, W1, b1, W2, b2, eps):
    blk = 1000
    grid = (N_NODES // blk,)
    row_spec = pl.BlockSpec((blk, D), lambda i: (i, 0))
    full_spec = pl.BlockSpec((D, D), lambda i: (0, 0))
    bias_spec = pl.BlockSpec((1, D), lambda i: (0, 0))
    return pl.pallas_call(
        _mlp_body,
        grid=grid,
        in_specs=[
            pl.BlockSpec(memory_space=pltpu.SMEM),
            row_spec, row_spec, row_spec,
            full_spec, bias_spec, full_spec, bias_spec,
        ],
        out_specs=row_spec,
        out_shape=jax.ShapeDtypeStruct((N_NODES, D), jnp.float32),
    )(eps.reshape(1, 1), x, p0, p1, W1, b1.reshape(1, D), W2, b2.reshape(1, D))


def kernel(x, edge_index, W1, b1, W2, b2, eps):
    src = edge_index[0].astype(jnp.int32)
    dst = edge_index[1].astype(jnp.int32)
    pad = E_PAD - N_EDGES
    packed = jnp.left_shift(dst, 16) | src
    k = jnp.arange(pad, dtype=jnp.int32)
    trash = ((N_NODES + k % (AGG_ROWS - N_NODES)) << 16) | (k % N_NODES)
    packed = jnp.concatenate([packed, trash]).reshape(NW, NCHUNK, CHUNK)
    zeros = jnp.zeros((AGG_ROWS, D), jnp.float32)

    p = _sc_aggregate(x, packed, zeros)
    return _mlp(x, p[0, :N_NODES], p[1, :N_NODES], W1, b1, W2, b2, eps)


# speedup vs baseline: 3.4255x; 1.0757x over previous
import functools

import numpy as np

import jax
import jax.numpy as jnp
from jax import lax
from jax.experimental import pallas as pl
from jax.experimental.pallas import tpu as pltpu
from jax.experimental.pallas import tpu_sc as plsc

N_NODES = 10000
N_EDGES = 320000
D = 128

NC = 2
NS = 16
NW = NC * NS

CHUNK = 128
NCHUNK = 80
E_TILE = CHUNK * NCHUNK
E_PAD = E_TILE * NW

AGG_ROWS = 10112
ROWS_PER_TILE = AGG_ROWS // NS


def _unpack_chunk(pk_v, j, src_b, dst_b):
    for r in range(CHUNK // 16):
        v = pk_v[j, pl.ds(16 * r, 16)]
        src_b[pl.ds(16 * r, 16)] = lax.bitwise_and(v, 0xFFFF)
        dst_b[pl.ds(16 * r, 16)] = lax.shift_right_logical(v, 16)


def _sc_agg_body(x_hbm, pk_hbm, z_hbm, p_hbm,
                 pk_v, srcb0, dstb0, srcb1, dstb1, gbuf0, gbuf1, agg,
                 sem0, sem1):
    cid = lax.axis_index("c")
    sid = lax.axis_index("s")
    wid = cid * NS + sid

    with jax.named_scope("sc_init"):
        pltpu.sync_copy(z_hbm.at[pl.ds(sid * ROWS_PER_TILE, ROWS_PER_TILE)],
                        agg.at[pl.ds(sid * ROWS_PER_TILE, ROWS_PER_TILE)])
        pltpu.sync_copy(pk_hbm.at[wid], pk_v)
        plsc.subcore_barrier()

    _unpack_chunk(pk_v, 0, srcb0, dstb0)
    pltpu.async_copy(x_hbm.at[srcb0], gbuf0, sem0)

    def body(i, carry):
        j = 2 * i
        jn = jnp.minimum(j + 2, NCHUNK - 1)
        _unpack_chunk(pk_v, j + 1, srcb1, dstb1)
        pltpu.async_copy(x_hbm.at[srcb1], gbuf1, sem1)
        pltpu.make_async_copy(x_hbm.at[srcb0], gbuf0, sem0).wait()
        pltpu.sync_copy(gbuf0, agg.at[dstb0], add=True)
        _unpack_chunk(pk_v, jn, srcb0, dstb0)
        pltpu.async_copy(x_hbm.at[srcb0], gbuf0, sem0)
        pltpu.make_async_copy(x_hbm.at[srcb1], gbuf1, sem1).wait()
        pltpu.sync_copy(gbuf1, agg.at[dstb1], add=True)
        return carry

    with jax.named_scope("sc_edge_loop"):
        lax.fori_loop(0, NCHUNK // 2, body, 0)
        pltpu.make_async_copy(x_hbm.at[srcb0], gbuf0, sem0).wait()
        plsc.subcore_barrier()

    with jax.named_scope("sc_writeback"):
        pltpu.sync_copy(agg.at[pl.ds(sid * ROWS_PER_TILE, ROWS_PER_TILE)],
                        p_hbm.at[cid, pl.ds(sid * ROWS_PER_TILE, ROWS_PER_TILE)])


def _sc_aggregate(x, packed, zeros):
    mesh = plsc.VectorSubcoreMesh(core_axis_name="c", subcore_axis_name="s")
    f = functools.partial(
        pl.kernel,
        mesh=mesh,
        out_type=jax.ShapeDtypeStruct((NC, AGG_ROWS, D), jnp.float32),
        scratch_types=[
            pltpu.VMEM((NCHUNK, CHUNK), jnp.int32),
            pltpu.VMEM((CHUNK,), jnp.int32),
            pltpu.VMEM((CHUNK,), jnp.int32),
            pltpu.VMEM((CHUNK,), jnp.int32),
            pltpu.VMEM((CHUNK,), jnp.int32),
            pltpu.VMEM((CHUNK, D), jnp.float32),
            pltpu.VMEM((CHUNK, D), jnp.float32),
            pltpu.VMEM_SHARED((AGG_ROWS, D), jnp.float32),
            pltpu.SemaphoreType.DMA,
            pltpu.SemaphoreType.DMA,
        ],
    )(_sc_agg_body)
    return f(x, packed, zeros)


def _mlp_body(eps_ref, x_ref, p0_ref, p1_ref, w1_ref, b1_ref, w2_ref, b2_ref,
              o_ref):
    h = ((1.0 + eps_ref[0, 0]) * x_ref[...]
         + p0_ref[0, :, :] + p1_ref[0, :, :])
    h = jnp.dot(h, w1_ref[...], preferred_element_type=jnp.float32) + b1_ref[...]
    h = jnp.maximum(h, 0.0)
    h = jnp.dot(h, w2_ref[...], preferred_element_type=jnp.float32) + b2_ref[...]
    o_ref[...] = jnp.maximum(h, 0.0)


def _mlp(x, p, W1, b1, W2, b2, eps):
    blk = 2000
    grid = (N_NODES // blk,)
    row_spec = pl.BlockSpec((blk, D), lambda i: (i, 0))
    p0_spec = pl.BlockSpec((1, blk, D), lambda i: (0, i, 0))
    p1_spec = pl.BlockSpec((1, blk, D), lambda i: (1, i, 0))
    full_spec = pl.BlockSpec((D, D), lambda i: (0, 0))
    bias_spec = pl.BlockSpec((1, D), lambda i: (0, 0))
    return pl.pallas_call(
        _mlp_body,
        grid=grid,
        in_specs=[
            pl.BlockSpec(memory_space=pltpu.SMEM),
            row_spec, p0_spec, p1_spec,
            full_spec, bias_spec, full_spec, bias_spec,
        ],
        out_specs=row_spec,
        out_shape=jax.ShapeDtypeStruct((N_NODES, D), jnp.float32),
    )(eps.reshape(1, 1), x, p, p, W1, b1.reshape(1, D), W2, b2.reshape(1, D))


def kernel(x, edge_index, W1, b1, W2, b2, eps):
    src = edge_index[0].astype(jnp.int32)
    dst = edge_index[1].astype(jnp.int32)
    pad = E_PAD - N_EDGES
    packed = jnp.left_shift(dst, 16) | src
    k = np.arange(pad, dtype=np.int64)
    trash = jnp.asarray(
        (((N_NODES + k % (AGG_ROWS - N_NODES)) << 16) | (k % N_NODES))
        .astype(np.int32))
    packed = jnp.concatenate([packed, trash]).reshape(NW, NCHUNK, CHUNK)
    zeros = jnp.asarray(np.zeros((AGG_ROWS, D), np.float32))

    p = _sc_aggregate(x, packed, zeros)
    return _mlp(x, p, W1, b1, W2, b2, eps)


# speedup vs baseline: 3.4625x; 1.0108x over previous
import functools

import numpy as np

import jax
import jax.numpy as jnp
from jax import lax
from jax.experimental import pallas as pl
from jax.experimental.pallas import tpu as pltpu
from jax.experimental.pallas import tpu_sc as plsc

N_NODES = 10000
N_EDGES = 320000
D = 128

NC = 2
NS = 16
NW = NC * NS

CHUNK = 128
NCHUNK = 80
E_TILE = CHUNK * NCHUNK
E_PAD = E_TILE * NW

AGG_ROWS = 10112
ROWS_PER_TILE = AGG_ROWS // NS


def _unpack_chunk(pk_v, j, src_b, dst_b):
    for r in range(CHUNK // 16):
        v = pk_v[pl.ds(j * CHUNK + 16 * r, 16)]
        src_b[pl.ds(16 * r, 16)] = lax.bitwise_and(v, 0xFFFF)
        dst_b[pl.ds(16 * r, 16)] = lax.shift_right_logical(v, 16)


def _sc_agg_body(x_hbm, pk_hbm, z_hbm, p_hbm,
                 pk_v, srcb0, dstb0, srcb1, dstb1, gbuf0, gbuf1, agg,
                 sem0, sem1):
    cid = lax.axis_index("c")
    sid = lax.axis_index("s")
    wid = cid * NS + sid

    with jax.named_scope("sc_init"):
        pltpu.sync_copy(pk_hbm.at[pl.ds(wid * E_TILE, E_TILE)], pk_v)
        _unpack_chunk(pk_v, 0, srcb0, dstb0)
        pltpu.async_copy(x_hbm.at[srcb0], gbuf0, sem0)
        _unpack_chunk(pk_v, 1, srcb1, dstb1)
        pltpu.async_copy(x_hbm.at[srcb1], gbuf1, sem1)
        pltpu.sync_copy(z_hbm.at[pl.ds(sid * ROWS_PER_TILE, ROWS_PER_TILE)],
                        agg.at[pl.ds(sid * ROWS_PER_TILE, ROWS_PER_TILE)])
        plsc.subcore_barrier()

    def body(i, carry):
        j = 2 * i
        pltpu.make_async_copy(x_hbm.at[srcb0], gbuf0, sem0).wait()
        pltpu.sync_copy(gbuf0, agg.at[dstb0], add=True)
        _unpack_chunk(pk_v, jnp.minimum(j + 2, NCHUNK - 1), srcb0, dstb0)
        pltpu.async_copy(x_hbm.at[srcb0], gbuf0, sem0)
        pltpu.make_async_copy(x_hbm.at[srcb1], gbuf1, sem1).wait()
        pltpu.sync_copy(gbuf1, agg.at[dstb1], add=True)
        _unpack_chunk(pk_v, jnp.minimum(j + 3, NCHUNK - 1), srcb1, dstb1)
        pltpu.async_copy(x_hbm.at[srcb1], gbuf1, sem1)
        return carry

    with jax.named_scope("sc_edge_loop"):
        lax.fori_loop(0, NCHUNK // 2, body, 0)
        pltpu.make_async_copy(x_hbm.at[srcb0], gbuf0, sem0).wait()
        pltpu.make_async_copy(x_hbm.at[srcb1], gbuf1, sem1).wait()
        plsc.subcore_barrier()

    with jax.named_scope("sc_writeback"):
        pltpu.sync_copy(agg.at[pl.ds(sid * ROWS_PER_TILE, ROWS_PER_TILE)],
                        p_hbm.at[cid, pl.ds(sid * ROWS_PER_TILE, ROWS_PER_TILE)])


def _sc_aggregate(x, packed, zeros):
    mesh = plsc.VectorSubcoreMesh(core_axis_name="c", subcore_axis_name="s")
    f = functools.partial(
        pl.kernel,
        mesh=mesh,
        out_type=jax.ShapeDtypeStruct((NC, AGG_ROWS, D), jnp.float32),
        scratch_types=[
            pltpu.VMEM((E_TILE,), jnp.int32),
            pltpu.VMEM((CHUNK,), jnp.int32),
            pltpu.VMEM((CHUNK,), jnp.int32),
            pltpu.VMEM((CHUNK,), jnp.int32),
            pltpu.VMEM((CHUNK,), jnp.int32),
            pltpu.VMEM((CHUNK, D), jnp.float32),
            pltpu.VMEM((CHUNK, D), jnp.float32),
            pltpu.VMEM_SHARED((AGG_ROWS, D), jnp.float32),
            pltpu.SemaphoreType.DMA,
            pltpu.SemaphoreType.DMA,
        ],
    )(_sc_agg_body)
    return f(x, packed, zeros)


def _mlp_body(eps_ref, x_ref, p0_ref, p1_ref, w1_ref, b1_ref, w2_ref, b2_ref,
              o_ref):
    h = ((1.0 + eps_ref[0, 0]) * x_ref[...]
         + p0_ref[0, :, :] + p1_ref[0, :, :])
    h = jnp.dot(h, w1_ref[...], preferred_element_type=jnp.float32) + b1_ref[...]
    h = jnp.maximum(h, 0.0)
    h = jnp.dot(h, w2_ref[...], preferred_element_type=jnp.float32) + b2_ref[...]
    o_ref[...] = jnp.maximum(h, 0.0)


def _mlp(x, p, W1, b1, W2, b2, eps):
    blk = 5000
    grid = (N_NODES // blk,)
    row_spec = pl.BlockSpec((blk, D), lambda i: (i, 0))
    p0_spec = pl.BlockSpec((1, blk, D), lambda i: (0, i, 0))
    p1_spec = pl.BlockSpec((1, blk, D), lambda i: (1, i, 0))
    full_spec = pl.BlockSpec((D, D), lambda i: (0, 0))
    bias_spec = pl.BlockSpec((1, D), lambda i: (0, 0))
    return pl.pallas_call(
        _mlp_body,
        grid=grid,
        in_specs=[
            pl.BlockSpec(memory_space=pltpu.SMEM),
            row_spec, p0_spec, p1_spec,
            full_spec, bias_spec, full_spec, bias_spec,
        ],
        out_specs=row_spec,
        out_shape=jax.ShapeDtypeStruct((N_NODES, D), jnp.float32),
    )(eps.reshape(1, 1), x, p, p, W1, b1.reshape(1, D), W2, b2.reshape(1, D))


def kernel(x, edge_index, W1, b1, W2, b2, eps):
    src = edge_index[0].astype(jnp.int32)
    dst = edge_index[1].astype(jnp.int32)
    pad = E_PAD - N_EDGES
    packed = jnp.left_shift(dst, 16) | src
    k = np.arange(pad, dtype=np.int64)
    trash = jnp.asarray(
        (((N_NODES + k % (AGG_ROWS - N_NODES)) << 16) | (k % N_NODES))
        .astype(np.int32))
    packed = jnp.concatenate([packed, trash])
    zeros = jnp.asarray(np.zeros((AGG_ROWS, D), np.float32))

    p = _sc_aggregate(x, packed, zeros)
    return _mlp(x, p, W1, b1, W2, b2, eps)
